# Initial kernel scaffold; baseline (speedup 1.0000x reference)
import jax
import jax.numpy as jnp
from jax.experimental import pallas as pl


def kernel(seg1, len1, seg2, len2):
    raise NotImplementedError("write your pallas kernel here")


# speedup vs baseline: 2.4279x; 2.4279x over previous
import functools

import jax
import jax.numpy as jnp
from jax import lax
from jax.experimental import pallas as pl
from jax.experimental.pallas import tpu as pltpu
from jax.experimental.pallas import tpu_sc as plsc

B = 16
L = 4096
S = 4096
NUM_SPECIAL = 4
BUDGET = S - NUM_SPECIAL
START_VALUE = 0
END_VALUE = 2
PAD_VALUE = 1

_LANES = 16
_HALF = S // 2
_CHUNKS = _HALF // _LANES

_mesh = plsc.VectorSubcoreMesh(core_axis_name="c", subcore_axis_name="s")


@functools.partial(
    pl.kernel,
    mesh=_mesh,
    out_type=jax.ShapeDtypeStruct((B, S), jnp.int32),
    scratch_types=[
        pltpu.VMEM((L,), jnp.int32),
        pltpu.VMEM((L,), jnp.int32),
        pltpu.VMEM((_HALF,), jnp.int32),
        pltpu.VMEM((_LANES,), jnp.int32),
        pltpu.VMEM((_LANES,), jnp.int32),
        pltpu.VMEM((3 * _LANES,), jnp.int32),
        pltpu.VMEM((3 * _LANES,), jnp.int32),
    ],
)
def _pack(seg1_hbm, len1_hbm, seg2_hbm, len2_hbm, out_hbm,
          s1_v, s2_v, o_v, l1_v, l2_v, b1_v, b2_v):
    wid = lax.axis_index("s") * 2 + lax.axis_index("c")
    row = wid // 2
    half = wid % 2
    c0 = half * _HALF

    pltpu.sync_copy(len1_hbm, l1_v)
    pltpu.sync_copy(len2_hbm, l2_v)
    pltpu.sync_copy(seg1_hbm.at[row], s1_v)
    pltpu.sync_copy(seg2_hbm.at[row], s2_v)

    b1_v[pl.ds(0, _LANES)] = l1_v[...]
    b2_v[pl.ds(0, _LANES)] = l2_v[...]
    l1 = b1_v[pl.ds(row, _LANES)][0]
    l2 = b2_v[pl.ds(row, _LANES)][0]

    t1 = jnp.maximum(jnp.minimum(l1, BUDGET - jnp.minimum(l2, BUDGET // 2)), 0)
    t2 = jnp.maximum(jnp.minimum(l2, BUDGET - t1), 0)

    iota = lax.iota(jnp.int32, _LANES)
    base = iota + c0
    pad = jnp.full((_LANES,), PAD_VALUE, jnp.int32)
    end = jnp.full((_LANES,), END_VALUE, jnp.int32)
    start = jnp.full((_LANES,), START_VALUE, jnp.int32)

    def shifted_window(seg_v, bounce_v, startu):
        st = jnp.clip(startu, 0, L - _LANES)
        bounce_v[pl.ds(_LANES, _LANES)] = seg_v[pl.ds(st, _LANES)]
        d = jnp.clip(startu - st, -_LANES, _LANES)
        return bounce_v[pl.ds(_LANES + d, _LANES)]

    def body(j, _):
        p0 = c0 + j * _LANES
        pos = base + j * _LANES
        g1 = shifted_window(s1_v, b1_v, p0 - 1)
        g2 = shifted_window(s2_v, b2_v, p0 - 3 - t1)

        out = jnp.where((pos >= 3 + t1) & (pos < 3 + t1 + t2), g2, pad)
        out = jnp.where((pos >= 1) & (pos < 1 + t1), g1, out)
        is_end = (pos == 1 + t1) | (pos == 2 + t1) | (pos == 3 + t1 + t2)
        out = jnp.where(is_end, end, out)
        out = jnp.where(pos == 0, start, out)
        o_v[pl.ds(j * _LANES, _LANES)] = out
        return 0

    lax.fori_loop(0, _CHUNKS, body, 0)

    pltpu.sync_copy(o_v, out_hbm.at[row, pl.ds(c0, _HALF)])


def kernel(seg1, len1, seg2, len2):
    return _pack(seg1.astype(jnp.int32), len1.astype(jnp.int32),
                 seg2.astype(jnp.int32), len2.astype(jnp.int32))


# speedup vs baseline: 2.6628x; 1.0968x over previous
import functools

import jax
import jax.numpy as jnp
from jax import lax
from jax.experimental import pallas as pl
from jax.experimental.pallas import tpu as pltpu
from jax.experimental.pallas import tpu_sc as plsc

B = 16
L = 4096
S = 4096
NUM_SPECIAL = 4
BUDGET = S - NUM_SPECIAL
START_VALUE = 0
END_VALUE = 2
PAD_VALUE = 1

_LANES = 16
_HALF = S // 2
_CHUNKS = _HALF // _LANES

_mesh = plsc.VectorSubcoreMesh(core_axis_name="c", subcore_axis_name="s")


@functools.partial(
    pl.kernel,
    mesh=_mesh,
    out_type=jax.ShapeDtypeStruct((B, S), jnp.int32),
    scratch_types=[
        pltpu.VMEM((L,), jnp.int32),
        pltpu.VMEM((L,), jnp.int32),
        pltpu.VMEM((_HALF,), jnp.int32),
        pltpu.VMEM((_LANES,), jnp.int32),
        pltpu.VMEM((_LANES,), jnp.int32),
        pltpu.VMEM((3 * _LANES,), jnp.int32),
        pltpu.VMEM((3 * _LANES,), jnp.int32),
        pltpu.SemaphoreType.DMA,
        pltpu.SemaphoreType.DMA,
        pltpu.SemaphoreType.DMA,
    ],
)
def _pack(seg1_hbm, len1_hbm, seg2_hbm, len2_hbm, out_hbm,
          s1_v, s2_v, o_v, l1_v, l2_v, b1_v, b2_v, sem1, sem2, seml):
    wid = lax.axis_index("s") * 2 + lax.axis_index("c")
    row = wid // 2
    half = wid % 2
    c0 = half * _HALF
    jlo = half * _CHUNKS
    jhi = jlo + _CHUNKS

    cl1 = pltpu.async_copy(len1_hbm, l1_v, seml)
    cl2 = pltpu.async_copy(len2_hbm, l2_v, seml)
    cs1 = pltpu.async_copy(seg1_hbm.at[row], s1_v, sem1)
    cs2 = pltpu.async_copy(seg2_hbm.at[row], s2_v, sem2)
    cl1.wait()
    cl2.wait()

    b1_v[pl.ds(0, _LANES)] = l1_v[...]
    b2_v[pl.ds(0, _LANES)] = l2_v[...]
    l1 = b1_v[pl.ds(row, _LANES)][0]
    l2 = b2_v[pl.ds(row, _LANES)][0]

    t1 = jnp.maximum(jnp.minimum(l1, BUDGET - jnp.minimum(l2, BUDGET // 2)), 0)
    t2 = jnp.maximum(jnp.minimum(l2, BUDGET - t1), 0)

    ja = (1 + t1) // _LANES
    jb = (3 + t1) // _LANES
    jc = (3 + t1 + t2) // _LANES

    iota = lax.iota(jnp.int32, _LANES)
    pad = jnp.full((_LANES,), PAD_VALUE, jnp.int32)
    end = jnp.full((_LANES,), END_VALUE, jnp.int32)
    start = jnp.full((_LANES,), START_VALUE, jnp.int32)

    cs1.wait()
    cs2.wait()

    lo1 = jnp.maximum(jlo, 1)
    hi1 = jnp.maximum(jnp.minimum(jhi, ja), lo1)

    @plsc.parallel_loop(lo1, hi1, unroll=8)
    def _(j):
        o_v[pl.ds(j * _LANES - c0, _LANES)] = s1_v[pl.ds(j * _LANES - 1, _LANES)]

    lo2 = jnp.maximum(jlo, jb + 1)
    hi2 = jnp.maximum(jnp.minimum(jhi, jc), lo2)
    sh2 = 3 + t1

    @plsc.parallel_loop(lo2, hi2, unroll=8)
    def _(j):
        o_v[pl.ds(j * _LANES - c0, _LANES)] = s2_v[pl.ds(j * _LANES - sh2, _LANES)]

    lo3 = jnp.maximum(jlo, jc + 1)
    hi3 = jnp.maximum(jhi, lo3)

    @plsc.parallel_loop(lo3, hi3, unroll=8)
    def _(j):
        o_v[pl.ds(j * _LANES - c0, _LANES)] = pad

    def shifted_window(seg_v, bounce_v, startu):
        st = jnp.clip(startu, 0, L - _LANES)
        bounce_v[pl.ds(_LANES, _LANES)] = seg_v[pl.ds(st, _LANES)]
        d = jnp.clip(startu - st, -_LANES, _LANES)
        return bounce_v[pl.ds(_LANES + d, _LANES)]

    def fix_chunk(j):
        @pl.when((j >= jlo) & (j < jhi))
        def _():
            p0 = j * _LANES
            pos = iota + p0
            g1 = shifted_window(s1_v, b1_v, p0 - 1)
            g2 = shifted_window(s2_v, b2_v, p0 - 3 - t1)

            out = jnp.where((pos >= 3 + t1) & (pos < 3 + t1 + t2), g2, pad)
            out = jnp.where((pos >= 1) & (pos < 1 + t1), g1, out)
            is_end = (pos == 1 + t1) | (pos == 2 + t1) | (pos == 3 + t1 + t2)
            out = jnp.where(is_end, end, out)
            out = jnp.where(pos == 0, start, out)
            o_v[pl.ds(p0 - c0, _LANES)] = out

    fix_chunk(jnp.int32(0))
    fix_chunk(ja)
    fix_chunk(jb)
    fix_chunk(jc)

    pltpu.sync_copy(o_v, out_hbm.at[row, pl.ds(c0, _HALF)])


def kernel(seg1, len1, seg2, len2):
    return _pack(seg1.astype(jnp.int32), len1.astype(jnp.int32),
                 seg2.astype(jnp.int32), len2.astype(jnp.int32))


# speedup vs baseline: 2.7016x; 1.0146x over previous
import functools

import jax
import jax.numpy as jnp
from jax import lax
from jax.experimental import pallas as pl
from jax.experimental.pallas import tpu as pltpu
from jax.experimental.pallas import tpu_sc as plsc

B = 16
L = 4096
S = 4096
NUM_SPECIAL = 4
BUDGET = S - NUM_SPECIAL
START_VALUE = 0
END_VALUE = 2
PAD_VALUE = 1

_LANES = 16
_HALF = S // 2
_CHUNKS = _HALF // _LANES

_mesh = plsc.VectorSubcoreMesh(core_axis_name="c", subcore_axis_name="s")


@functools.partial(
    pl.kernel,
    mesh=_mesh,
    out_type=jax.ShapeDtypeStruct((B, S), jnp.int32),
    scratch_types=[
        pltpu.VMEM((L,), jnp.int32),
        pltpu.VMEM((L,), jnp.int32),
        pltpu.VMEM((_HALF,), jnp.int32),
        pltpu.VMEM((_LANES,), jnp.int32),
        pltpu.VMEM((_LANES,), jnp.int32),
        pltpu.VMEM((3 * _LANES,), jnp.int32),
        pltpu.VMEM((3 * _LANES,), jnp.int32),
        pltpu.SemaphoreType.DMA,
        pltpu.SemaphoreType.DMA,
        pltpu.SemaphoreType.DMA,
    ],
)
def _pack(seg1_hbm, len1_hbm, seg2_hbm, len2_hbm, out_hbm,
          s1_v, s2_v, o_v, l1_v, l2_v, b1_v, b2_v, sem1, sem2, seml):
    wid = lax.axis_index("s") * 2 + lax.axis_index("c")
    row = wid // 2
    half = wid % 2
    c0 = half * _HALF
    jlo = half * _CHUNKS
    jhi = jlo + _CHUNKS

    cl1 = pltpu.async_copy(len1_hbm, l1_v, seml)
    cl2 = pltpu.async_copy(len2_hbm, l2_v, seml)
    cs1 = pltpu.async_copy(seg1_hbm.at[row], s1_v, sem1)
    cs2 = pltpu.async_copy(seg2_hbm.at[row], s2_v, sem2)
    cl1.wait()
    cl2.wait()

    b1_v[pl.ds(0, _LANES)] = l1_v[...]
    b2_v[pl.ds(0, _LANES)] = l2_v[...]
    l1 = b1_v[pl.ds(row, _LANES)][0]
    l2 = b2_v[pl.ds(row, _LANES)][0]

    t1 = jnp.maximum(jnp.minimum(l1, BUDGET - jnp.minimum(l2, BUDGET // 2)), 0)
    t2 = jnp.maximum(jnp.minimum(l2, BUDGET - t1), 0)

    ja = (1 + t1) // _LANES
    jb = (3 + t1) // _LANES
    jc = (3 + t1 + t2) // _LANES

    iota = lax.iota(jnp.int32, _LANES)
    pad = jnp.full((_LANES,), PAD_VALUE, jnp.int32)
    end = jnp.full((_LANES,), END_VALUE, jnp.int32)
    start = jnp.full((_LANES,), START_VALUE, jnp.int32)

    cs1.wait()
    cs2.wait()

    lo1 = jnp.maximum(jlo, 1)
    hi1 = jnp.maximum(jnp.minimum(jhi, ja), lo1)

    @plsc.parallel_loop(lo1, hi1, unroll=4)
    def _(j):
        o_v[pl.ds(j * _LANES - c0, _LANES)] = s1_v[pl.ds(j * _LANES - 1, _LANES)]

    lo2 = jnp.maximum(jlo, jb + 1)
    hi2 = jnp.maximum(jnp.minimum(jhi, jc), lo2)
    sh2 = 3 + t1

    @plsc.parallel_loop(lo2, hi2, unroll=4)
    def _(j):
        o_v[pl.ds(j * _LANES - c0, _LANES)] = s2_v[pl.ds(j * _LANES - sh2, _LANES)]

    lo3 = jnp.maximum(jlo, jc + 1)
    hi3 = jnp.maximum(jhi, lo3)

    @plsc.parallel_loop(lo3, hi3, unroll=4)
    def _(j):
        o_v[pl.ds(j * _LANES - c0, _LANES)] = pad

    def shifted_window(seg_v, bounce_v, startu):
        st = jnp.clip(startu, 0, L - _LANES)
        bounce_v[pl.ds(_LANES, _LANES)] = seg_v[pl.ds(st, _LANES)]
        d = jnp.clip(startu - st, -_LANES, _LANES)
        return bounce_v[pl.ds(_LANES + d, _LANES)]

    def fix_chunk(k, carry):
        j = jnp.where(k == 0, 0, jnp.where(k == 1, ja, jnp.where(k == 2, jb, jc)))

        @pl.when((j >= jlo) & (j < jhi))
        def _():
            p0 = j * _LANES
            pos = iota + p0
            g1 = shifted_window(s1_v, b1_v, p0 - 1)
            g2 = shifted_window(s2_v, b2_v, p0 - 3 - t1)

            out = jnp.where((pos >= 3 + t1) & (pos < 3 + t1 + t2), g2, pad)
            out = jnp.where((pos >= 1) & (pos < 1 + t1), g1, out)
            is_end = (pos == 1 + t1) | (pos == 2 + t1) | (pos == 3 + t1 + t2)
            out = jnp.where(is_end, end, out)
            out = jnp.where(pos == 0, start, out)
            o_v[pl.ds(p0 - c0, _LANES)] = out

        return carry

    lax.fori_loop(0, 4, fix_chunk, 0)

    pltpu.sync_copy(o_v, out_hbm.at[row, pl.ds(c0, _HALF)])


def kernel(seg1, len1, seg2, len2):
    return _pack(seg1.astype(jnp.int32), len1.astype(jnp.int32),
                 seg2.astype(jnp.int32), len2.astype(jnp.int32))
